# Initial kernel scaffold; baseline (speedup 1.0000x reference)
import jax
import jax.numpy as jnp
from jax.experimental import pallas as pl


def kernel(points, features, npoint):
    raise NotImplementedError("write your pallas kernel here")


# speedup vs baseline: 2.2851x; 2.2851x over previous
import jax
import jax.numpy as jnp
from jax import lax
from jax.experimental import pallas as pl

_N = 4096
_NPOINT = 512
_R2 = 0.25
_BIG = 1e10
_PREC = lax.Precision.DEFAULT
_TB = 128


def _body(x_ref, xt_ref, p_ref, pt_ref, aafc_ref, aafr_ref,
          aapc_ref, aapr_ref, out_ref):
    f32 = jnp.float32
    i32 = jnp.int32
    iota = lax.broadcasted_iota(i32, (1, _N), 1)
    iota_out = lax.broadcasted_iota(i32, (1, 2 * _NPOINT), 1)

    pt = pt_ref[0]
    aapr = aapr_ref[0]
    cnt = jnp.zeros((1, _N), f32)
    for j in range(_N // _TB):
        pblk = p_ref[0, pl.ds(j * _TB, _TB), :]
        aac = aapc_ref[0, pl.ds(j * _TB, _TB), :]
        ab = jnp.dot(pblk, pt, precision=_PREC)
        sq = (aac + aapr) - 2.0 * ab
        cnt = cnt + jnp.sum((sq < _R2).astype(f32), axis=0, keepdims=True)
    weight = 1.0 / cnt

    xt = xt_ref[0]
    aafr = aafr_ref[0]

    def ffps_step(t, carry):
        mind, last, acc = carry
        xl = x_ref[0, pl.ds(last, 1), :]
        aal = aafc_ref[0, pl.ds(last, 1), :]
        ab = jnp.dot(xl, xt, precision=_PREC)
        d = (aafr + aal) - 2.0 * ab
        mind = jnp.minimum(mind, d)
        mx = jnp.max(mind)
        nxt = jnp.min(jnp.where(mind == mx, iota, _N)).astype(i32)
        acc = jnp.where(iota_out == (t + 1), nxt, acc)
        return mind, nxt, acc

    acc0 = jnp.zeros((1, 2 * _NPOINT), i32)
    mind0 = jnp.full((1, _N), _BIG, f32)
    _, _, acc = lax.fori_loop(0, _NPOINT - 1, ffps_step,
                              (mind0, jnp.int32(0), acc0))

    def dfps_step(t, carry):
        mind, last, acc = carry
        lp = p_ref[0, pl.ds(last, 1), :]
        a0 = jnp.abs(pt[0:1, :] - lp[:, 0:1])
        a1 = jnp.abs(pt[1:2, :] - lp[:, 1:2])
        a2 = jnp.abs(pt[2:3, :] - lp[:, 2:3])
        d = (a0 + a1) + a2
        mind = jnp.minimum(mind, d)
        prod = mind * weight
        mx = jnp.max(prod)
        nxt = jnp.min(jnp.where(prod == mx, iota, _N)).astype(i32)
        acc = jnp.where(iota_out == (_NPOINT + t + 1), nxt, acc)
        return mind, nxt, acc

    _, _, acc = lax.fori_loop(0, _NPOINT - 1, dfps_step,
                              (mind0, jnp.int32(0), acc))
    out_ref[0] = acc


def kernel(points, features, npoint):
    B, N, _ = points.shape
    f32 = jnp.float32
    ffps = jnp.concatenate([points, jnp.swapaxes(features, 1, 2)], axis=2)
    aaf = jnp.sum(ffps * ffps, axis=-1, keepdims=True)
    aap = jnp.sum(points * points, axis=-1, keepdims=True)
    xt = jnp.swapaxes(ffps, 1, 2)
    pt = jnp.swapaxes(points, 1, 2)
    aaf_row = jnp.swapaxes(aaf, 1, 2)
    aap_row = jnp.swapaxes(aap, 1, 2)

    spec3 = lambda s: pl.BlockSpec((1,) + s, lambda b: (b, 0, 0))
    out = pl.pallas_call(
        _body,
        grid=(B,),
        in_specs=[
            spec3((N, ffps.shape[2])),
            spec3((ffps.shape[2], N)),
            spec3((N, 3)),
            spec3((3, N)),
            spec3((N, 1)),
            spec3((1, N)),
            spec3((N, 1)),
            spec3((1, N)),
        ],
        out_specs=spec3((1, 2 * _NPOINT)),
        out_shape=jax.ShapeDtypeStruct((B, 1, 2 * _NPOINT), jnp.int32),
    )(ffps.astype(f32), xt.astype(f32), points.astype(f32), pt.astype(f32),
      aaf.astype(f32), aaf_row.astype(f32), aap.astype(f32),
      aap_row.astype(f32))
    return out.reshape(B, 2 * _NPOINT)


# speedup vs baseline: 3.6026x; 1.5765x over previous
import jax
import jax.numpy as jnp
from jax import lax
from jax.experimental import pallas as pl

_N = 4096
_NPOINT = 512
_R2 = 0.25
_BIG = 1e10
_PREC = lax.Precision.DEFAULT
_TB = 128
_B = 2


def _body(x_ref, xt_ref, p_ref, pt_ref, aafc_ref, aafr_ref,
          aapc_ref, aapr_ref, out_ref):
    f32 = jnp.float32
    i32 = jnp.int32
    iota = lax.broadcasted_iota(i32, (1, _N), 1)
    iota_out = lax.broadcasted_iota(i32, (1, 2 * _NPOINT), 1)

    weights = []
    for b in range(_B):
        pt = pt_ref[b]
        aapr = aapr_ref[b]
        cnt = jnp.zeros((1, _N), f32)
        for j in range(_N // _TB):
            pblk = p_ref[b, pl.ds(j * _TB, _TB), :]
            aac = aapc_ref[b, pl.ds(j * _TB, _TB), :]
            ab = jnp.dot(pblk, pt, precision=_PREC)
            sq = (aac + aapr) - 2.0 * ab
            cnt = cnt + jnp.sum((sq < _R2).astype(f32), axis=0, keepdims=True)
        weights.append(1.0 / cnt)

    def argmax_first(v):
        mx = jnp.max(v)
        return jnp.min(jnp.where(v == mx, iota, _N)).astype(i32)

    def step(t, carry):
        mf, lf, md, ld, acc = carry
        nmf, nlf, nmd, nld, nacc = [], [], [], [], []
        for b in range(_B):
            xl = x_ref[b, pl.ds(lf[b], 1), :]
            aal = aafc_ref[b, pl.ds(lf[b], 1), :]
            ab = jnp.dot(xl, xt_ref[b], precision=_PREC)
            d = (aafr_ref[b] + aal) - 2.0 * ab
            mfb = jnp.minimum(mf[b], d)
            nxf = argmax_first(mfb)
            lp = p_ref[b, pl.ds(ld[b], 1), :]
            pt = pt_ref[b]
            a0 = jnp.abs(pt[0:1, :] - lp[:, 0:1])
            a1 = jnp.abs(pt[1:2, :] - lp[:, 1:2])
            a2 = jnp.abs(pt[2:3, :] - lp[:, 2:3])
            dm = (a0 + a1) + a2
            mdb = jnp.minimum(md[b], dm)
            nxd = argmax_first(mdb * weights[b])
            accb = jnp.where(iota_out == (t + 1), nxf, acc[b])
            accb = jnp.where(iota_out == (_NPOINT + t + 1), nxd, accb)
            nmf.append(mfb); nlf.append(nxf)
            nmd.append(mdb); nld.append(nxd)
            nacc.append(accb)
        return nmf, nlf, nmd, nld, nacc

    mind0 = jnp.full((1, _N), _BIG, f32)
    acc0 = jnp.zeros((1, 2 * _NPOINT), i32)
    zero = jnp.int32(0)
    init = ([mind0] * _B, [zero] * _B, [mind0] * _B, [zero] * _B,
            [acc0] * _B)
    _, _, _, _, acc = lax.fori_loop(0, _NPOINT - 1, step, init)
    for b in range(_B):
        out_ref[b] = acc[b]


def kernel(points, features, npoint):
    B, N, _ = points.shape
    f32 = jnp.float32
    ffps = jnp.concatenate([points, jnp.swapaxes(features, 1, 2)], axis=2)
    aaf = jnp.sum(ffps * ffps, axis=-1, keepdims=True)
    aap = jnp.sum(points * points, axis=-1, keepdims=True)
    xt = jnp.swapaxes(ffps, 1, 2)
    pt = jnp.swapaxes(points, 1, 2)
    aaf_row = jnp.swapaxes(aaf, 1, 2)
    aap_row = jnp.swapaxes(aap, 1, 2)

    out = pl.pallas_call(
        _body,
        out_shape=jax.ShapeDtypeStruct((B, 1, 2 * _NPOINT), jnp.int32),
    )(ffps.astype(f32), xt.astype(f32), points.astype(f32), pt.astype(f32),
      aaf.astype(f32), aaf_row.astype(f32), aap.astype(f32),
      aap_row.astype(f32))
    return out.reshape(B, 2 * _NPOINT)
